# Initial kernel scaffold; baseline (speedup 1.0000x reference)
import jax
import jax.numpy as jnp
from jax.experimental import pallas as pl


def kernel(x_in, W_qkv, W_dw, W_po, temperature, attn1, attn2, attn3, attn4):
    raise NotImplementedError("write your pallas kernel here")


# speedup vs baseline: 2.8846x; 2.8846x over previous
import functools

import jax
import jax.numpy as jnp
from jax.experimental import pallas as pl
from jax.experimental.pallas import tpu as pltpu

HEADS = 6


def _a1_body(x_ref, w_ref, o_ref):
    o_ref[...] = jax.lax.dot_general(
        w_ref[...].astype(jnp.bfloat16), x_ref[...].astype(jnp.bfloat16),
        (((1,), (0,)), ((), ())),
        preferred_element_type=jnp.float32)


def _dwconv_flat(x, wdw, w_img):
    x = x.astype(jnp.bfloat16).astype(jnp.float32)
    c, hw = x.shape
    col = jax.lax.broadcasted_iota(jnp.int32, (1, hw), 1) % w_img
    mask_l = (col != 0).astype(x.dtype)
    mask_r = (col != (w_img - 1)).astype(x.dtype)
    zero_cache = {}

    def shifted(s):
        if s == 0:
            return x
        if s > 0:
            if s not in zero_cache:
                zero_cache[s] = jnp.zeros((c, s), x.dtype)
            return jnp.concatenate([x[:, s:], zero_cache[s]], axis=1)
        if -s not in zero_cache:
            zero_cache[-s] = jnp.zeros((c, -s), x.dtype)
        return jnp.concatenate([zero_cache[-s], x[:, :s]], axis=1)

    out = None
    for di in (-1, 0, 1):
        for dj in (-1, 0, 1):
            t = wdw[:, (di + 1) * 3 + (dj + 1)][:, None] * shifted(di * w_img + dj)
            if dj == 1:
                t = t * mask_r
            elif dj == -1:
                t = t * mask_l
            out = t if out is None else out + t
    return out


def _dw_tile(w_img, halo, cur_ref, prev_ref, nxt_ref, wdw_ref):
    tt = pl.program_id(2)
    ntt = pl.num_programs(2)
    nt = cur_ref.shape[1]
    mp = jnp.where(tt > 0, 1.0, 0.0).astype(jnp.float32)
    mn = jnp.where(tt < ntt - 1, 1.0, 0.0).astype(jnp.float32)
    x_ext = jnp.concatenate(
        [prev_ref[...] * mp, cur_ref[...], nxt_ref[...] * mn], axis=1)
    dw = _dwconv_flat(x_ext, wdw_ref[...], w_img)
    return dw[:, halo:halo + nt]


def _a2a_body(w_img, halo, qk_ref, prev_ref, nxt_ref, wdw_ref,
              dw_ref, ssq_ref):
    qk = _dw_tile(w_img, halo, qk_ref, prev_ref, nxt_ref, wdw_ref)
    cc = qk.shape[0] // 2
    q = qk[:cc]
    k = qk[cc:]
    dw_ref[...] = qk
    s = jnp.concatenate(
        [jnp.sum(q * q, axis=1)[None, :], jnp.sum(k * k, axis=1)[None, :]],
        axis=0)
    tt = pl.program_id(2)

    @pl.when(tt == 0)
    def _():
        ssq_ref[...] = s

    @pl.when(tt > 0)
    def _():
        ssq_ref[...] += s


def _a2b_body(qk_ref, rn_ref, gram_ref):
    cc = qk_ref.shape[0] // 2
    qn = (qk_ref[:cc] * rn_ref[0]).astype(jnp.bfloat16)
    kn = (qk_ref[cc:] * rn_ref[1]).astype(jnp.bfloat16)
    g = jax.lax.dot_general(
        qn, kn, (((1,), (1,)), ((), ())), preferred_element_type=jnp.float32)
    tt = pl.program_id(2)

    @pl.when(tt == 0)
    def _():
        gram_ref[...] = g

    @pl.when(tt > 0)
    def _():
        gram_ref[...] += g


def _a2v_body(w_img, halo, v_ref, prev_ref, nxt_ref, wdw_ref, o_ref):
    o_ref[...] = _dw_tile(w_img, halo, v_ref, prev_ref, nxt_ref, wdw_ref)


def _spa_body(ks, gram_ref, ssq_ref, temp_ref, coef_ref, spa_ref):
    bh = gram_ref.shape[0]
    cc = gram_ref.shape[1]
    n = cc * cc
    nq = jnp.maximum(jnp.sqrt(ssq_ref[:, 0, :]), 1e-12)
    nk = jnp.maximum(jnp.sqrt(ssq_ref[:, 1, :]), 1e-12)
    attn = gram_ref[...] / (nq[:, :, None] * nk[:, None, :]) * temp_ref[...]

    bits = jax.lax.bitcast_convert_type(attn, jnp.int32)
    skey = jnp.where(bits < 0, bits ^ jnp.int32(0x7FFFFFFF), bits)

    klist = (n * 1 // 2, n * 2 // 3, n * 3 // 4, n * 4 // 5)
    m = jnp.max(attn, axis=(1, 2), keepdims=True)
    e = jnp.exp(attn - m)
    coeff = jnp.zeros_like(attn)
    for ki, kk in enumerate(klist):
        def bit_step(i, t_u, kk=kk):
            b = 31 - i
            t_try = t_u | (jnp.int32(1) << b)
            t_cmp = t_try ^ jnp.int32(-0x80000000)
            cnt = jnp.sum((skey >= t_cmp[:, :, None]).astype(jnp.int32),
                          axis=(1, 2), keepdims=True)[:, :, 0]
            return jnp.where(cnt >= kk, t_try, t_u)

        t_u = jax.lax.fori_loop(0, 32, bit_step,
                                jnp.zeros((bh, 1), jnp.int32))
        th_skey = t_u ^ jnp.int32(-0x80000000)
        th_bits = jnp.where(th_skey < 0, th_skey ^ jnp.int32(0x7FFFFFFF),
                            th_skey)
        th = jax.lax.bitcast_convert_type(th_bits, jnp.float32)
        th3 = th[:, :, None]
        gt = (attn > th3).astype(jnp.float32)
        eq = (attn == th3).astype(jnp.float32)
        c_g = jnp.sum(gt, axis=(1, 2), keepdims=True)
        c_e = jnp.sum(eq, axis=(1, 2), keepdims=True)
        e_th = jnp.exp(th3 - m)
        z = jnp.sum(e * gt, axis=(1, 2), keepdims=True) + (kk - c_g) * e_th
        a = coef_ref[0, ki]
        coeff = coeff + gt * (a / z) + eq * (a * (kk - c_g) / (c_e * z))
    spa_ref[...] = e * coeff


def _c_body(heads, gram_like_spa_ref, wpo_ref, v_ref, o_ref):
    spa = gram_like_spa_ref[...]
    cc = spa.shape[1]
    wpo = wpo_ref[...]
    cols = []
    for h in range(heads):
        cols.append(jax.lax.dot_general(
            wpo[:, h * cc:(h + 1) * cc].astype(jnp.bfloat16),
            spa[h].astype(jnp.bfloat16), (((1,), (0,)), ((), ())),
            preferred_element_type=jnp.float32))
    mmat = jnp.concatenate(cols, axis=1)
    o_ref[...] = jax.lax.dot_general(
        mmat.astype(jnp.bfloat16), v_ref[...].astype(jnp.bfloat16),
        (((1,), (0,)), ((), ())),
        preferred_element_type=jnp.float32)


def _pick_tiles(hw):
    for nt in (14, 8, 7, 4, 2):
        if hw % nt == 0:
            return nt
    return 1


def _plan_dw_tiles(hw, w_img):
    halo = w_img
    while halo % 128 != 0 or halo <= w_img:
        halo += w_img
    best = None
    for ntile in range(1, 64):
        if hw % ntile:
            continue
        nt = hw // ntile
        if nt % halo:
            continue
        if nt * 64 * 4 <= 4 * 1024 * 1024 or best is None:
            best = (ntile, nt, halo)
            if nt * 64 * 4 <= 4 * 1024 * 1024:
                return best
    return best


def kernel(x_in, W_qkv, W_dw, W_po, temperature, attn1, attn2, attn3, attn4):
    b, dim, h_img, w_img = x_in.shape
    heads = HEADS
    cc = dim // heads
    hw = h_img * w_img
    f32 = jnp.float32

    base = jnp.arange(cc)
    perm = []
    for h in range(heads):
        perm.append(h * cc + base)
        perm.append(dim + h * cc + base)
    for h in range(heads):
        perm.append(2 * dim + h * cc + base)
    perm = jnp.concatenate(perm)

    w1 = W_qkv[:, :, 0, 0][perm]
    wdw = W_dw[:, 0].reshape(3 * dim, 9)[perm]
    wpo = W_po[:, :, 0, 0]

    x = x_in.reshape(b, dim, hw)
    ntile = _pick_tiles(hw)
    nt = hw // ntile

    qkv_pre = pl.pallas_call(
        _a1_body,
        grid=(b, ntile),
        in_specs=[
            pl.BlockSpec((None, dim, nt), lambda bb, tt: (bb, 0, tt)),
            pl.BlockSpec((3 * dim, dim), lambda bb, tt: (0, 0)),
        ],
        out_specs=pl.BlockSpec((None, 3 * dim, nt), lambda bb, tt: (bb, 0, tt)),
        out_shape=jax.ShapeDtypeStruct((b, 3 * dim, hw), f32),
    )(x, w1)

    ntile2, nt2, halo = _plan_dw_tiles(hw, w_img)
    rr = nt2 // halo
    nhalo = hw // halo

    def _prev_idx(bb, hh, tt):
        return (bb, hh, jnp.maximum(tt * rr - 1, 0))

    def _nxt_idx(bb, hh, tt):
        return (bb, hh, jnp.minimum((tt + 1) * rr, nhalo - 1))

    qkdw, ssq = pl.pallas_call(
        functools.partial(_a2a_body, w_img, halo),
        grid=(b, heads, ntile2),
        in_specs=[
            pl.BlockSpec((None, 2 * cc, nt2), lambda bb, hh, tt: (bb, hh, tt)),
            pl.BlockSpec((None, 2 * cc, halo), _prev_idx),
            pl.BlockSpec((None, 2 * cc, halo), _nxt_idx),
            pl.BlockSpec((2 * cc, 9), lambda bb, hh, tt: (hh, 0)),
        ],
        out_specs=[
            pl.BlockSpec((None, 2 * cc, nt2), lambda bb, hh, tt: (bb, hh, tt)),
            pl.BlockSpec((None, 2, cc),
                         lambda bb, hh, tt: (bb * heads + hh, 0, 0)),
        ],
        out_shape=[
            jax.ShapeDtypeStruct((b, 2 * cc * heads, hw), f32),
            jax.ShapeDtypeStruct((b * heads, 2, cc), f32),
        ],
    )(qkv_pre, qkv_pre, qkv_pre, wdw)

    rnorm = (1.0 / jnp.maximum(jnp.sqrt(ssq), 1e-12))[..., None]

    gram = pl.pallas_call(
        _a2b_body,
        grid=(b, heads, ntile2),
        in_specs=[
            pl.BlockSpec((None, 2 * cc, nt2), lambda bb, hh, tt: (bb, hh, tt)),
            pl.BlockSpec((None, 2, cc, 1),
                         lambda bb, hh, tt: (bb * heads + hh, 0, 0, 0)),
        ],
        out_specs=pl.BlockSpec((None, cc, cc),
                               lambda bb, hh, tt: (bb * heads + hh, 0, 0)),
        out_shape=jax.ShapeDtypeStruct((b * heads, cc, cc), f32),
    )(qkdw, rnorm)

    def _prev_idx_v(bb, hh, tt):
        return (bb, 2 * heads + hh, jnp.maximum(tt * rr - 1, 0))

    def _nxt_idx_v(bb, hh, tt):
        return (bb, 2 * heads + hh, jnp.minimum((tt + 1) * rr, nhalo - 1))

    v = pl.pallas_call(
        functools.partial(_a2v_body, w_img, halo),
        grid=(b, heads, ntile2),
        in_specs=[
            pl.BlockSpec((None, cc, nt2),
                         lambda bb, hh, tt: (bb, 2 * heads + hh, tt)),
            pl.BlockSpec((None, cc, halo), _prev_idx_v),
            pl.BlockSpec((None, cc, halo), _nxt_idx_v),
            pl.BlockSpec((cc, 9), lambda bb, hh, tt: (2 * heads + hh, 0)),
        ],
        out_specs=pl.BlockSpec((None, None, cc, nt2),
                               lambda bb, hh, tt: (bb, hh, 0, tt)),
        out_shape=jax.ShapeDtypeStruct((b, heads, cc, hw), f32),
    )(qkv_pre, qkv_pre, qkv_pre, wdw)

    temp_b = jnp.broadcast_to(temperature[None, :, :, :],
                              (b, heads, 1, 1)).reshape(b * heads, 1, 1)
    coef = jnp.concatenate([attn1, attn2, attn3, attn4]).reshape(1, 4)
    spa = pl.pallas_call(
        functools.partial(_spa_body, None),
        in_specs=[
            pl.BlockSpec((b * heads, cc, cc), lambda: (0, 0, 0)),
            pl.BlockSpec((b * heads, 2, cc), lambda: (0, 0, 0)),
            pl.BlockSpec((b * heads, 1, 1), lambda: (0, 0, 0)),
            pl.BlockSpec((1, 4), lambda: (0, 0)),
        ],
        out_specs=pl.BlockSpec((b * heads, cc, cc), lambda: (0, 0, 0)),
        out_shape=jax.ShapeDtypeStruct((b * heads, cc, cc), f32),
    )(gram, jnp.ones((b * heads, 2, cc), f32), temp_b, coef)

    v2 = v.reshape(b, dim, hw)
    spa_b = spa.reshape(b, heads, cc, cc)
    out = pl.pallas_call(
        functools.partial(_c_body, heads),
        grid=(b, ntile),
        in_specs=[
            pl.BlockSpec((None, heads, cc, cc), lambda bb, tt: (bb, 0, 0, 0)),
            pl.BlockSpec((dim, dim), lambda bb, tt: (0, 0)),
            pl.BlockSpec((None, dim, nt), lambda bb, tt: (bb, 0, tt)),
        ],
        out_specs=pl.BlockSpec((None, dim, nt), lambda bb, tt: (bb, 0, tt)),
        out_shape=jax.ShapeDtypeStruct((b, dim, hw), f32),
    )(spa_b, wpo, v2)

    return out.reshape(b, dim, h_img, w_img)


# speedup vs baseline: 2.8956x; 1.0038x over previous
import functools

import jax
import jax.numpy as jnp
from jax import lax
from jax.experimental import pallas as pl
from jax.experimental.pallas import tpu as pltpu
from jax.experimental.pallas import tpu_sc as plsc

HEADS = 6


def _a1_body(x_ref, w_ref, o_ref):
    o_ref[...] = jax.lax.dot_general(
        w_ref[...].astype(jnp.bfloat16), x_ref[...].astype(jnp.bfloat16),
        (((1,), (0,)), ((), ())),
        preferred_element_type=jnp.float32)


def _dwconv_flat(x, wdw, w_img):
    x = x.astype(jnp.bfloat16).astype(jnp.float32)
    c, hw = x.shape
    col = jax.lax.broadcasted_iota(jnp.int32, (1, hw), 1) % w_img
    mask_l = (col != 0).astype(x.dtype)
    mask_r = (col != (w_img - 1)).astype(x.dtype)
    zero_cache = {}

    def shifted(s):
        if s == 0:
            return x
        if s > 0:
            if s not in zero_cache:
                zero_cache[s] = jnp.zeros((c, s), x.dtype)
            return jnp.concatenate([x[:, s:], zero_cache[s]], axis=1)
        if -s not in zero_cache:
            zero_cache[-s] = jnp.zeros((c, -s), x.dtype)
        return jnp.concatenate([zero_cache[-s], x[:, :s]], axis=1)

    out = None
    for di in (-1, 0, 1):
        for dj in (-1, 0, 1):
            t = wdw[:, (di + 1) * 3 + (dj + 1)][:, None] * shifted(di * w_img + dj)
            if dj == 1:
                t = t * mask_r
            elif dj == -1:
                t = t * mask_l
            out = t if out is None else out + t
    return out


def _dw_tile(w_img, halo, cur_ref, prev_ref, nxt_ref, wdw_ref):
    tt = pl.program_id(2)
    ntt = pl.num_programs(2)
    nt = cur_ref.shape[1]
    mp = jnp.where(tt > 0, 1.0, 0.0).astype(jnp.float32)
    mn = jnp.where(tt < ntt - 1, 1.0, 0.0).astype(jnp.float32)
    x_ext = jnp.concatenate(
        [prev_ref[...] * mp, cur_ref[...], nxt_ref[...] * mn], axis=1)
    dw = _dwconv_flat(x_ext, wdw_ref[...], w_img)
    return dw[:, halo:halo + nt]


def _a2a_body(w_img, halo, qk_ref, prev_ref, nxt_ref, wdw_ref,
              dw_ref, ssq_ref):
    qk = _dw_tile(w_img, halo, qk_ref, prev_ref, nxt_ref, wdw_ref)
    cc = qk.shape[0] // 2
    q = qk[:cc]
    k = qk[cc:]
    dw_ref[...] = qk
    s = jnp.concatenate(
        [jnp.sum(q * q, axis=1)[None, :], jnp.sum(k * k, axis=1)[None, :]],
        axis=0)
    tt = pl.program_id(2)

    @pl.when(tt == 0)
    def _():
        ssq_ref[...] = s

    @pl.when(tt > 0)
    def _():
        ssq_ref[...] += s


def _a2b_body(qk_ref, rn_ref, gram_ref):
    cc = qk_ref.shape[0] // 2
    qn = (qk_ref[:cc] * rn_ref[0]).astype(jnp.bfloat16)
    kn = (qk_ref[cc:] * rn_ref[1]).astype(jnp.bfloat16)
    g = jax.lax.dot_general(
        qn, kn, (((1,), (1,)), ((), ())), preferred_element_type=jnp.float32)
    tt = pl.program_id(2)

    @pl.when(tt == 0)
    def _():
        gram_ref[...] = g

    @pl.when(tt > 0)
    def _():
        gram_ref[...] += g


def _a2v_body(w_img, halo, v_ref, prev_ref, nxt_ref, wdw_ref, o_ref):
    o_ref[...] = _dw_tile(w_img, halo, v_ref, prev_ref, nxt_ref, wdw_ref)


def _spa_body(ks, gram_ref, ssq_ref, temp_ref, coef_ref, spa_ref):
    bh = gram_ref.shape[0]
    cc = gram_ref.shape[1]
    n = cc * cc
    nq = jnp.maximum(jnp.sqrt(ssq_ref[:, 0, :]), 1e-12)
    nk = jnp.maximum(jnp.sqrt(ssq_ref[:, 1, :]), 1e-12)
    attn = gram_ref[...] / (nq[:, :, None] * nk[:, None, :]) * temp_ref[...]

    bits = jax.lax.bitcast_convert_type(attn, jnp.int32)
    skey = jnp.where(bits < 0, bits ^ jnp.int32(0x7FFFFFFF), bits)

    klist = (n * 1 // 2, n * 2 // 3, n * 3 // 4, n * 4 // 5)
    m = jnp.max(attn, axis=(1, 2), keepdims=True)
    e = jnp.exp(attn - m)
    coeff = jnp.zeros_like(attn)
    for ki, kk in enumerate(klist):
        def bit_step(i, t_u, kk=kk):
            b = 31 - i
            t_try = t_u | (jnp.int32(1) << b)
            t_cmp = t_try ^ jnp.int32(-0x80000000)
            cnt = jnp.sum((skey >= t_cmp[:, :, None]).astype(jnp.int32),
                          axis=(1, 2), keepdims=True)[:, :, 0]
            return jnp.where(cnt >= kk, t_try, t_u)

        t_u = jax.lax.fori_loop(0, 32, bit_step,
                                jnp.zeros((bh, 1), jnp.int32))
        th_skey = t_u ^ jnp.int32(-0x80000000)
        th_bits = jnp.where(th_skey < 0, th_skey ^ jnp.int32(0x7FFFFFFF),
                            th_skey)
        th = jax.lax.bitcast_convert_type(th_bits, jnp.float32)
        th3 = th[:, :, None]
        gt = (attn > th3).astype(jnp.float32)
        eq = (attn == th3).astype(jnp.float32)
        c_g = jnp.sum(gt, axis=(1, 2), keepdims=True)
        c_e = jnp.sum(eq, axis=(1, 2), keepdims=True)
        e_th = jnp.exp(th3 - m)
        z = jnp.sum(e * gt, axis=(1, 2), keepdims=True) + (kk - c_g) * e_th
        a = coef_ref[0, ki]
        coeff = coeff + gt * (a / z) + eq * (a * (kk - c_g) / (c_e * z))
    spa_ref[...] = e * coeff


def _xl_reduce(v, op):
    for sh in (8, 4, 2, 1):
        idx = lax.iota(jnp.int32, 16) ^ sh
        v = op(v, v.at[idx].get(mode="promise_in_bounds"))
    return v


def _make_spa_sc(bh, n, klist):
    nv = n // 16
    mesh = plsc.VectorSubcoreMesh(core_axis_name="c", subcore_axis_name="s")

    @functools.partial(
        pl.kernel, mesh=mesh,
        out_type=jax.ShapeDtypeStruct((bh, n), jnp.float32),
        scratch_types=[
            pltpu.VMEM((n,), jnp.float32),
            pltpu.VMEM((n,), jnp.int32),
            pltpu.VMEM((n,), jnp.float32),
            pltpu.VMEM((64,), jnp.float32),
        ],
    )
    def spa_sc(attn_hbm, skey_hbm, coef_hbm, out_hbm,
               attn_v, skey_v, out_v, coef_v):
        i32 = jnp.int32
        wid = lax.axis_index("s") * 2 + lax.axis_index("c")

        @pl.when(wid < bh)
        def _():
            pltpu.sync_copy(attn_hbm.at[wid], attn_v)
            pltpu.sync_copy(skey_hbm.at[wid], skey_v)
            pltpu.sync_copy(coef_hbm, coef_v)
            coefs = [coef_v[pl.ds(ki * 16, 16)] for ki in range(4)]

            def pass1(i, mv):
                return jnp.maximum(mv, attn_v[pl.ds(i * 16, 16)])

            mv = lax.fori_loop(0, nv, pass1,
                               jnp.full((16,), -3.4e38, jnp.float32))
            m_s = _xl_reduce(mv, jnp.maximum)

            ths = []
            for kk in klist:
                def bit_step(bi, t_u, kk=kk):
                    b = 31 - bi
                    t_try = t_u | (i32(1) << b)
                    t_cmp = t_try ^ i32(-0x80000000)

                    def count(i, cnt):
                        sv = skey_v[pl.ds(i * 16, 16)]
                        return cnt + jnp.where(sv >= t_cmp, 1, 0)

                    cnt = _xl_reduce(
                        lax.fori_loop(0, nv, count, jnp.zeros((16,), i32)),
                        jnp.add)
                    return jnp.where(cnt >= kk, t_try, t_u)

                t_u = lax.fori_loop(0, 32, bit_step, jnp.zeros((16,), i32))
                ths.append(t_u ^ i32(-0x80000000))

            def pass2(i, acc):
                a = attn_v[pl.ds(i * 16, 16)]
                sv = skey_v[pl.ds(i * 16, 16)]
                e = jnp.exp(a - m_s)
                out = []
                for ki in range(4):
                    z, zeq, cg, ce = acc[ki]
                    gt = sv > ths[ki]
                    eq = sv == ths[ki]
                    out.append((z + jnp.where(gt, e, 0.0),
                                zeq + jnp.where(eq, e, 0.0),
                                cg + jnp.where(gt, 1, 0),
                                ce + jnp.where(eq, 1, 0)))
                return tuple(out)

            zero = (jnp.zeros((16,), jnp.float32),
                    jnp.zeros((16,), jnp.float32),
                    jnp.zeros((16,), i32), jnp.zeros((16,), i32))
            acc = lax.fori_loop(0, nv, pass2, (zero, zero, zero, zero))

            wks = []
            for ki, kk in enumerate(klist):
                z, zeq, cg, ce = acc[ki]
                z_tot = _xl_reduce(z, jnp.add)
                zeq_tot = _xl_reduce(zeq, jnp.add)
                cgf = _xl_reduce(cg, jnp.add).astype(jnp.float32)
                cef = _xl_reduce(ce, jnp.add).astype(jnp.float32)
                e_th = zeq_tot / cef
                zz = z_tot + (kk - cgf) * e_th
                wks.append((coefs[ki] / zz,
                            coefs[ki] * (kk - cgf) / (cef * zz)))

            def pass3(i, carry):
                a = attn_v[pl.ds(i * 16, 16)]
                sv = skey_v[pl.ds(i * 16, 16)]
                e = jnp.exp(a - m_s)
                coeff = jnp.zeros((16,), jnp.float32)
                for ki in range(4):
                    coeff = coeff + jnp.where(sv > ths[ki], wks[ki][0], 0.0)
                    coeff = coeff + jnp.where(sv == ths[ki], wks[ki][1], 0.0)
                out_v[pl.ds(i * 16, 16)] = e * coeff
                return carry

            lax.fori_loop(0, nv, pass3, 0)
            pltpu.sync_copy(out_v, out_hbm.at[wid])

    return spa_sc


def _c_body(heads, gram_like_spa_ref, wpo_ref, v_ref, o_ref):
    spa = gram_like_spa_ref[...]
    cc = spa.shape[1]
    wpo = wpo_ref[...]
    cols = []
    for h in range(heads):
        cols.append(jax.lax.dot_general(
            wpo[:, h * cc:(h + 1) * cc].astype(jnp.bfloat16),
            spa[h].astype(jnp.bfloat16), (((1,), (0,)), ((), ())),
            preferred_element_type=jnp.float32))
    mmat = jnp.concatenate(cols, axis=1)
    o_ref[...] = jax.lax.dot_general(
        mmat.astype(jnp.bfloat16), v_ref[...].astype(jnp.bfloat16),
        (((1,), (0,)), ((), ())),
        preferred_element_type=jnp.float32)


def _pick_tiles(hw):
    for nt in (14, 8, 7, 4, 2):
        if hw % nt == 0:
            return nt
    return 1


def _plan_dw_tiles(hw, w_img):
    halo = w_img
    while halo % 128 != 0 or halo <= w_img:
        halo += w_img
    best = None
    for ntile in range(1, 64):
        if hw % ntile:
            continue
        nt = hw // ntile
        if nt % halo:
            continue
        if nt * 64 * 4 <= 4 * 1024 * 1024 or best is None:
            best = (ntile, nt, halo)
            if nt * 64 * 4 <= 4 * 1024 * 1024:
                return best
    return best


def kernel(x_in, W_qkv, W_dw, W_po, temperature, attn1, attn2, attn3, attn4):
    b, dim, h_img, w_img = x_in.shape
    heads = HEADS
    cc = dim // heads
    hw = h_img * w_img
    f32 = jnp.float32

    base = jnp.arange(cc)
    perm = []
    for h in range(heads):
        perm.append(h * cc + base)
        perm.append(dim + h * cc + base)
    for h in range(heads):
        perm.append(2 * dim + h * cc + base)
    perm = jnp.concatenate(perm)

    w1 = W_qkv[:, :, 0, 0][perm]
    wdw = W_dw[:, 0].reshape(3 * dim, 9)[perm]
    wpo = W_po[:, :, 0, 0]

    x = x_in.reshape(b, dim, hw)
    ntile = _pick_tiles(hw)
    nt = hw // ntile

    qkv_pre = pl.pallas_call(
        _a1_body,
        grid=(b, ntile),
        in_specs=[
            pl.BlockSpec((None, dim, nt), lambda bb, tt: (bb, 0, tt)),
            pl.BlockSpec((3 * dim, dim), lambda bb, tt: (0, 0)),
        ],
        out_specs=pl.BlockSpec((None, 3 * dim, nt), lambda bb, tt: (bb, 0, tt)),
        out_shape=jax.ShapeDtypeStruct((b, 3 * dim, hw), f32),
    )(x, w1)

    ntile2, nt2, halo = _plan_dw_tiles(hw, w_img)
    rr = nt2 // halo
    nhalo = hw // halo

    def _prev_idx(bb, hh, tt):
        return (bb, hh, jnp.maximum(tt * rr - 1, 0))

    def _nxt_idx(bb, hh, tt):
        return (bb, hh, jnp.minimum((tt + 1) * rr, nhalo - 1))

    qkdw, ssq = pl.pallas_call(
        functools.partial(_a2a_body, w_img, halo),
        grid=(b, heads, ntile2),
        in_specs=[
            pl.BlockSpec((None, 2 * cc, nt2), lambda bb, hh, tt: (bb, hh, tt)),
            pl.BlockSpec((None, 2 * cc, halo), _prev_idx),
            pl.BlockSpec((None, 2 * cc, halo), _nxt_idx),
            pl.BlockSpec((2 * cc, 9), lambda bb, hh, tt: (hh, 0)),
        ],
        out_specs=[
            pl.BlockSpec((None, 2 * cc, nt2), lambda bb, hh, tt: (bb, hh, tt)),
            pl.BlockSpec((None, 2, cc),
                         lambda bb, hh, tt: (bb * heads + hh, 0, 0)),
        ],
        out_shape=[
            jax.ShapeDtypeStruct((b, 2 * cc * heads, hw), f32),
            jax.ShapeDtypeStruct((b * heads, 2, cc), f32),
        ],
    )(qkv_pre, qkv_pre, qkv_pre, wdw)

    rnorm = (1.0 / jnp.maximum(jnp.sqrt(ssq), 1e-12))[..., None]

    gram = pl.pallas_call(
        _a2b_body,
        grid=(b, heads, ntile2),
        in_specs=[
            pl.BlockSpec((None, 2 * cc, nt2), lambda bb, hh, tt: (bb, hh, tt)),
            pl.BlockSpec((None, 2, cc, 1),
                         lambda bb, hh, tt: (bb * heads + hh, 0, 0, 0)),
        ],
        out_specs=pl.BlockSpec((None, cc, cc),
                               lambda bb, hh, tt: (bb * heads + hh, 0, 0)),
        out_shape=jax.ShapeDtypeStruct((b * heads, cc, cc), f32),
    )(qkdw, rnorm)

    def _prev_idx_v(bb, hh, tt):
        return (bb, 2 * heads + hh, jnp.maximum(tt * rr - 1, 0))

    def _nxt_idx_v(bb, hh, tt):
        return (bb, 2 * heads + hh, jnp.minimum((tt + 1) * rr, nhalo - 1))

    v = pl.pallas_call(
        functools.partial(_a2v_body, w_img, halo),
        grid=(b, heads, ntile2),
        in_specs=[
            pl.BlockSpec((None, cc, nt2),
                         lambda bb, hh, tt: (bb, 2 * heads + hh, tt)),
            pl.BlockSpec((None, cc, halo), _prev_idx_v),
            pl.BlockSpec((None, cc, halo), _nxt_idx_v),
            pl.BlockSpec((cc, 9), lambda bb, hh, tt: (2 * heads + hh, 0)),
        ],
        out_specs=pl.BlockSpec((None, None, cc, nt2),
                               lambda bb, hh, tt: (bb, hh, 0, tt)),
        out_shape=jax.ShapeDtypeStruct((b, heads, cc, hw), f32),
    )(qkv_pre, qkv_pre, qkv_pre, wdw)

    temp_b = jnp.broadcast_to(temperature[None, :, :, :],
                              (b, heads, 1, 1)).reshape(b * heads, 1, 1)
    coef = jnp.concatenate([attn1, attn2, attn3, attn4])
    attn_rows = (gram * temp_b).reshape(b * heads, cc * cc)
    klist = (cc * cc * 1 // 2, cc * cc * 2 // 3, cc * cc * 3 // 4,
             cc * cc * 4 // 5)
    coef_bc = jnp.broadcast_to(coef[:, None], (4, 16)).reshape(64)
    bits = jax.lax.bitcast_convert_type(attn_rows, jnp.int32)
    skey_rows = jnp.where(bits < 0, bits ^ jnp.int32(0x7FFFFFFF), bits)
    spa = _make_spa_sc(b * heads, cc * cc, klist)(
        attn_rows, skey_rows, coef_bc)

    v2 = v.reshape(b, dim, hw)
    spa_b = spa.reshape(b, heads, cc, cc)
    out = pl.pallas_call(
        functools.partial(_c_body, heads),
        grid=(b, ntile),
        in_specs=[
            pl.BlockSpec((None, heads, cc, cc), lambda bb, tt: (bb, 0, 0, 0)),
            pl.BlockSpec((dim, dim), lambda bb, tt: (0, 0)),
            pl.BlockSpec((None, dim, nt), lambda bb, tt: (bb, 0, tt)),
        ],
        out_specs=pl.BlockSpec((None, dim, nt), lambda bb, tt: (bb, 0, tt)),
        out_shape=jax.ShapeDtypeStruct((b, dim, hw), f32),
    )(spa_b, wpo, v2)

    return out.reshape(b, dim, h_img, w_img)


# speedup vs baseline: 3.0511x; 1.0537x over previous
import functools

import jax
import jax.numpy as jnp
from jax import lax
from jax.experimental import pallas as pl
from jax.experimental.pallas import tpu as pltpu
from jax.experimental.pallas import tpu_sc as plsc

HEADS = 6


def _a1_body(x_ref, w_ref, o_ref):
    o_ref[...] = jax.lax.dot_general(
        w_ref[...].astype(jnp.bfloat16), x_ref[...].astype(jnp.bfloat16),
        (((1,), (0,)), ((), ())),
        preferred_element_type=jnp.float32).astype(jnp.bfloat16)


def _dwconv_flat(x, wdw, w_img):
    x = x.astype(jnp.bfloat16).astype(jnp.float32)
    c, hw = x.shape
    col = jax.lax.broadcasted_iota(jnp.int32, (1, hw), 1) % w_img
    mask_l = (col != 0).astype(x.dtype)
    mask_r = (col != (w_img - 1)).astype(x.dtype)
    zero_cache = {}

    def shifted(s):
        if s == 0:
            return x
        if s > 0:
            if s not in zero_cache:
                zero_cache[s] = jnp.zeros((c, s), x.dtype)
            return jnp.concatenate([x[:, s:], zero_cache[s]], axis=1)
        if -s not in zero_cache:
            zero_cache[-s] = jnp.zeros((c, -s), x.dtype)
        return jnp.concatenate([zero_cache[-s], x[:, :s]], axis=1)

    out = None
    for di in (-1, 0, 1):
        for dj in (-1, 0, 1):
            t = wdw[:, (di + 1) * 3 + (dj + 1)][:, None] * shifted(di * w_img + dj)
            if dj == 1:
                t = t * mask_r
            elif dj == -1:
                t = t * mask_l
            out = t if out is None else out + t
    return out


def _dw_tile(w_img, halo, cur_ref, prev_ref, nxt_ref, wdw_ref):
    tt = pl.program_id(2)
    ntt = pl.num_programs(2)
    nt = cur_ref.shape[1]
    mp = jnp.where(tt > 0, 1.0, 0.0).astype(jnp.float32)
    mn = jnp.where(tt < ntt - 1, 1.0, 0.0).astype(jnp.float32)
    x_ext = jnp.concatenate(
        [prev_ref[...].astype(jnp.float32) * mp,
         cur_ref[...].astype(jnp.float32),
         nxt_ref[...].astype(jnp.float32) * mn], axis=1)
    dw = _dwconv_flat(x_ext, wdw_ref[...], w_img)
    return dw[:, halo:halo + nt]


def _a2a_body(w_img, halo, qk_ref, prev_ref, nxt_ref, wdw_ref,
              dw_ref, ssq_ref):
    qk = _dw_tile(w_img, halo, qk_ref, prev_ref, nxt_ref, wdw_ref)
    cc = qk.shape[0] // 2
    q = qk[:cc]
    k = qk[cc:]
    dw_ref[...] = qk
    s = jnp.concatenate(
        [jnp.sum(q * q, axis=1)[None, :], jnp.sum(k * k, axis=1)[None, :]],
        axis=0)
    tt = pl.program_id(2)

    @pl.when(tt == 0)
    def _():
        ssq_ref[...] = s

    @pl.when(tt > 0)
    def _():
        ssq_ref[...] += s


def _a2b_body(qk_ref, rn_ref, gram_ref):
    cc = qk_ref.shape[0] // 2
    qn = (qk_ref[:cc] * rn_ref[0]).astype(jnp.bfloat16)
    kn = (qk_ref[cc:] * rn_ref[1]).astype(jnp.bfloat16)
    g = jax.lax.dot_general(
        qn, kn, (((1,), (1,)), ((), ())), preferred_element_type=jnp.float32)
    tt = pl.program_id(2)

    @pl.when(tt == 0)
    def _():
        gram_ref[...] = g

    @pl.when(tt > 0)
    def _():
        gram_ref[...] += g


def _a2v_body(w_img, halo, v_ref, prev_ref, nxt_ref, wdw_ref, o_ref):
    o_ref[...] = _dw_tile(w_img, halo, v_ref, prev_ref, nxt_ref,
                          wdw_ref).astype(jnp.bfloat16)


def _spa_body(ks, gram_ref, ssq_ref, temp_ref, coef_ref, spa_ref):
    bh = gram_ref.shape[0]
    cc = gram_ref.shape[1]
    n = cc * cc
    nq = jnp.maximum(jnp.sqrt(ssq_ref[:, 0, :]), 1e-12)
    nk = jnp.maximum(jnp.sqrt(ssq_ref[:, 1, :]), 1e-12)
    attn = gram_ref[...] / (nq[:, :, None] * nk[:, None, :]) * temp_ref[...]

    bits = jax.lax.bitcast_convert_type(attn, jnp.int32)
    skey = jnp.where(bits < 0, bits ^ jnp.int32(0x7FFFFFFF), bits)

    klist = (n * 1 // 2, n * 2 // 3, n * 3 // 4, n * 4 // 5)
    m = jnp.max(attn, axis=(1, 2), keepdims=True)
    e = jnp.exp(attn - m)
    coeff = jnp.zeros_like(attn)
    for ki, kk in enumerate(klist):
        def bit_step(i, t_u, kk=kk):
            b = 31 - i
            t_try = t_u | (jnp.int32(1) << b)
            t_cmp = t_try ^ jnp.int32(-0x80000000)
            cnt = jnp.sum((skey >= t_cmp[:, :, None]).astype(jnp.int32),
                          axis=(1, 2), keepdims=True)[:, :, 0]
            return jnp.where(cnt >= kk, t_try, t_u)

        t_u = jax.lax.fori_loop(0, 32, bit_step,
                                jnp.zeros((bh, 1), jnp.int32))
        th_skey = t_u ^ jnp.int32(-0x80000000)
        th_bits = jnp.where(th_skey < 0, th_skey ^ jnp.int32(0x7FFFFFFF),
                            th_skey)
        th = jax.lax.bitcast_convert_type(th_bits, jnp.float32)
        th3 = th[:, :, None]
        gt = (attn > th3).astype(jnp.float32)
        eq = (attn == th3).astype(jnp.float32)
        c_g = jnp.sum(gt, axis=(1, 2), keepdims=True)
        c_e = jnp.sum(eq, axis=(1, 2), keepdims=True)
        e_th = jnp.exp(th3 - m)
        z = jnp.sum(e * gt, axis=(1, 2), keepdims=True) + (kk - c_g) * e_th
        a = coef_ref[0, ki]
        coeff = coeff + gt * (a / z) + eq * (a * (kk - c_g) / (c_e * z))
    spa_ref[...] = e * coeff


def _xl_reduce(v, op):
    for sh in (8, 4, 2, 1):
        idx = lax.iota(jnp.int32, 16) ^ sh
        v = op(v, v.at[idx].get(mode="promise_in_bounds"))
    return v


def _make_spa_sc(bh, n, klist):
    nv = n // 16
    mesh = plsc.VectorSubcoreMesh(core_axis_name="c", subcore_axis_name="s")

    @functools.partial(
        pl.kernel, mesh=mesh,
        out_type=jax.ShapeDtypeStruct((bh, n), jnp.float32),
        scratch_types=[
            pltpu.VMEM((n,), jnp.float32),
            pltpu.VMEM((n,), jnp.int32),
            pltpu.VMEM((n,), jnp.float32),
            pltpu.VMEM((64,), jnp.float32),
        ],
    )
    def spa_sc(attn_hbm, skey_hbm, coef_hbm, out_hbm,
               attn_v, skey_v, out_v, coef_v):
        i32 = jnp.int32
        wid = lax.axis_index("s") * 2 + lax.axis_index("c")

        @pl.when(wid < bh)
        def _():
            pltpu.sync_copy(attn_hbm.at[wid], attn_v)
            pltpu.sync_copy(skey_hbm.at[wid], skey_v)
            pltpu.sync_copy(coef_hbm, coef_v)
            coefs = [coef_v[pl.ds(ki * 16, 16)] for ki in range(4)]

            def pass1(i, mv):
                return jnp.maximum(mv, attn_v[pl.ds(i * 16, 16)])

            mv = lax.fori_loop(0, nv, pass1,
                               jnp.full((16,), -3.4e38, jnp.float32))
            m_s = _xl_reduce(mv, jnp.maximum)

            ths = []
            for kk in klist:
                def bit_step(bi, t_u, kk=kk):
                    b = 31 - bi
                    t_try = t_u | (i32(1) << b)
                    t_cmp = t_try ^ i32(-0x80000000)

                    def count(i, cnt):
                        sv = skey_v[pl.ds(i * 16, 16)]
                        return cnt + jnp.where(sv >= t_cmp, 1, 0)

                    cnt = _xl_reduce(
                        lax.fori_loop(0, nv, count, jnp.zeros((16,), i32)),
                        jnp.add)
                    return jnp.where(cnt >= kk, t_try, t_u)

                t_u = lax.fori_loop(0, 32, bit_step, jnp.zeros((16,), i32))
                ths.append(t_u ^ i32(-0x80000000))

            def pass2(i, acc):
                a = attn_v[pl.ds(i * 16, 16)]
                sv = skey_v[pl.ds(i * 16, 16)]
                e = jnp.exp(a - m_s)
                out = []
                for ki in range(4):
                    z, zeq, cg, ce = acc[ki]
                    gt = sv > ths[ki]
                    eq = sv == ths[ki]
                    out.append((z + jnp.where(gt, e, 0.0),
                                zeq + jnp.where(eq, e, 0.0),
                                cg + jnp.where(gt, 1, 0),
                                ce + jnp.where(eq, 1, 0)))
                return tuple(out)

            zero = (jnp.zeros((16,), jnp.float32),
                    jnp.zeros((16,), jnp.float32),
                    jnp.zeros((16,), i32), jnp.zeros((16,), i32))
            acc = lax.fori_loop(0, nv, pass2, (zero, zero, zero, zero))

            wks = []
            for ki, kk in enumerate(klist):
                z, zeq, cg, ce = acc[ki]
                z_tot = _xl_reduce(z, jnp.add)
                zeq_tot = _xl_reduce(zeq, jnp.add)
                cgf = _xl_reduce(cg, jnp.add).astype(jnp.float32)
                cef = _xl_reduce(ce, jnp.add).astype(jnp.float32)
                e_th = zeq_tot / cef
                zz = z_tot + (kk - cgf) * e_th
                wks.append((coefs[ki] / zz,
                            coefs[ki] * (kk - cgf) / (cef * zz)))

            def pass3(i, carry):
                a = attn_v[pl.ds(i * 16, 16)]
                sv = skey_v[pl.ds(i * 16, 16)]
                e = jnp.exp(a - m_s)
                coeff = jnp.zeros((16,), jnp.float32)
                for ki in range(4):
                    coeff = coeff + jnp.where(sv > ths[ki], wks[ki][0], 0.0)
                    coeff = coeff + jnp.where(sv == ths[ki], wks[ki][1], 0.0)
                out_v[pl.ds(i * 16, 16)] = e * coeff
                return carry

            lax.fori_loop(0, nv, pass3, 0)
            pltpu.sync_copy(out_v, out_hbm.at[wid])

    return spa_sc


def _c_body(heads, gram_like_spa_ref, wpo_ref, v_ref, o_ref):
    spa = gram_like_spa_ref[...]
    cc = spa.shape[1]
    wpo = wpo_ref[...]
    cols = []
    for h in range(heads):
        cols.append(jax.lax.dot_general(
            wpo[:, h * cc:(h + 1) * cc].astype(jnp.bfloat16),
            spa[h].astype(jnp.bfloat16), (((1,), (0,)), ((), ())),
            preferred_element_type=jnp.float32))
    mmat = jnp.concatenate(cols, axis=1)
    o_ref[...] = jax.lax.dot_general(
        mmat.astype(jnp.bfloat16), v_ref[...],
        (((1,), (0,)), ((), ())),
        preferred_element_type=jnp.float32)


def _pick_tiles(hw):
    for nt in (14, 8, 7, 4, 2):
        if hw % nt == 0:
            return nt
    return 1


def _plan_dw_tiles(hw, w_img):
    halo = w_img
    while halo % 128 != 0 or halo <= w_img:
        halo += w_img
    best = None
    for ntile in range(1, 64):
        if hw % ntile:
            continue
        nt = hw // ntile
        if nt % halo:
            continue
        if nt * 64 * 4 <= 4 * 1024 * 1024 or best is None:
            best = (ntile, nt, halo)
            if nt * 64 * 4 <= 4 * 1024 * 1024:
                return best
    return best


def kernel(x_in, W_qkv, W_dw, W_po, temperature, attn1, attn2, attn3, attn4):
    b, dim, h_img, w_img = x_in.shape
    heads = HEADS
    cc = dim // heads
    hw = h_img * w_img
    f32 = jnp.float32

    base = jnp.arange(cc)
    perm = []
    for h in range(heads):
        perm.append(h * cc + base)
        perm.append(dim + h * cc + base)
    for h in range(heads):
        perm.append(2 * dim + h * cc + base)
    perm = jnp.concatenate(perm)

    w1 = W_qkv[:, :, 0, 0][perm]
    wdw = W_dw[:, 0].reshape(3 * dim, 9)[perm]
    wpo = W_po[:, :, 0, 0]

    x = x_in.reshape(b, dim, hw)
    ntile = _pick_tiles(hw)
    nt = hw // ntile

    qkv_pre = pl.pallas_call(
        _a1_body,
        grid=(b, ntile),
        in_specs=[
            pl.BlockSpec((None, dim, nt), lambda bb, tt: (bb, 0, tt)),
            pl.BlockSpec((3 * dim, dim), lambda bb, tt: (0, 0)),
        ],
        out_specs=pl.BlockSpec((None, 3 * dim, nt), lambda bb, tt: (bb, 0, tt)),
        out_shape=jax.ShapeDtypeStruct((b, 3 * dim, hw), jnp.bfloat16),
    )(x, w1)

    ntile2, nt2, halo = _plan_dw_tiles(hw, w_img)
    rr = nt2 // halo
    nhalo = hw // halo

    def _prev_idx(bb, hh, tt):
        return (bb, hh, jnp.maximum(tt * rr - 1, 0))

    def _nxt_idx(bb, hh, tt):
        return (bb, hh, jnp.minimum((tt + 1) * rr, nhalo - 1))

    qkdw, ssq = pl.pallas_call(
        functools.partial(_a2a_body, w_img, halo),
        grid=(b, heads, ntile2),
        in_specs=[
            pl.BlockSpec((None, 2 * cc, nt2), lambda bb, hh, tt: (bb, hh, tt)),
            pl.BlockSpec((None, 2 * cc, halo), _prev_idx),
            pl.BlockSpec((None, 2 * cc, halo), _nxt_idx),
            pl.BlockSpec((2 * cc, 9), lambda bb, hh, tt: (hh, 0)),
        ],
        out_specs=[
            pl.BlockSpec((None, 2 * cc, nt2), lambda bb, hh, tt: (bb, hh, tt)),
            pl.BlockSpec((None, 2, cc),
                         lambda bb, hh, tt: (bb * heads + hh, 0, 0)),
        ],
        out_shape=[
            jax.ShapeDtypeStruct((b, 2 * cc * heads, hw), f32),
            jax.ShapeDtypeStruct((b * heads, 2, cc), f32),
        ],
    )(qkv_pre, qkv_pre, qkv_pre, wdw)

    rnorm = (1.0 / jnp.maximum(jnp.sqrt(ssq), 1e-12))[..., None]

    gram = pl.pallas_call(
        _a2b_body,
        grid=(b, heads, ntile2),
        in_specs=[
            pl.BlockSpec((None, 2 * cc, nt2), lambda bb, hh, tt: (bb, hh, tt)),
            pl.BlockSpec((None, 2, cc, 1),
                         lambda bb, hh, tt: (bb * heads + hh, 0, 0, 0)),
        ],
        out_specs=pl.BlockSpec((None, cc, cc),
                               lambda bb, hh, tt: (bb * heads + hh, 0, 0)),
        out_shape=jax.ShapeDtypeStruct((b * heads, cc, cc), f32),
    )(qkdw, rnorm)

    def _prev_idx_v(bb, hh, tt):
        return (bb, 2 * heads + hh, jnp.maximum(tt * rr - 1, 0))

    def _nxt_idx_v(bb, hh, tt):
        return (bb, 2 * heads + hh, jnp.minimum((tt + 1) * rr, nhalo - 1))

    v = pl.pallas_call(
        functools.partial(_a2v_body, w_img, halo),
        grid=(b, heads, ntile2),
        in_specs=[
            pl.BlockSpec((None, cc, nt2),
                         lambda bb, hh, tt: (bb, 2 * heads + hh, tt)),
            pl.BlockSpec((None, cc, halo), _prev_idx_v),
            pl.BlockSpec((None, cc, halo), _nxt_idx_v),
            pl.BlockSpec((cc, 9), lambda bb, hh, tt: (2 * heads + hh, 0)),
        ],
        out_specs=pl.BlockSpec((None, None, cc, nt2),
                               lambda bb, hh, tt: (bb, hh, 0, tt)),
        out_shape=jax.ShapeDtypeStruct((b, heads, cc, hw), jnp.bfloat16),
    )(qkv_pre, qkv_pre, qkv_pre, wdw)

    temp_b = jnp.broadcast_to(temperature[None, :, :, :],
                              (b, heads, 1, 1)).reshape(b * heads, 1, 1)
    coef = jnp.concatenate([attn1, attn2, attn3, attn4])
    attn_rows = (gram * temp_b).reshape(b * heads, cc * cc)
    klist = (cc * cc * 1 // 2, cc * cc * 2 // 3, cc * cc * 3 // 4,
             cc * cc * 4 // 5)
    coef_bc = jnp.broadcast_to(coef[:, None], (4, 16)).reshape(64)
    bits = jax.lax.bitcast_convert_type(attn_rows, jnp.int32)
    skey_rows = jnp.where(bits < 0, bits ^ jnp.int32(0x7FFFFFFF), bits)
    spa = _make_spa_sc(b * heads, cc * cc, klist)(
        attn_rows, skey_rows, coef_bc)

    v2 = v.reshape(b, dim, hw)
    spa_b = spa.reshape(b, heads, cc, cc)
    out = pl.pallas_call(
        functools.partial(_c_body, heads),
        grid=(b, ntile),
        in_specs=[
            pl.BlockSpec((None, heads, cc, cc), lambda bb, tt: (bb, 0, 0, 0)),
            pl.BlockSpec((dim, dim), lambda bb, tt: (0, 0)),
            pl.BlockSpec((None, dim, nt), lambda bb, tt: (bb, 0, tt)),
        ],
        out_specs=pl.BlockSpec((None, dim, nt), lambda bb, tt: (bb, 0, tt)),
        out_shape=jax.ShapeDtypeStruct((b, dim, hw), f32),
    )(spa_b, wpo, v2)

    return out.reshape(b, dim, h_img, w_img)


# speedup vs baseline: 3.0695x; 1.0060x over previous
import functools

import jax
import jax.numpy as jnp
from jax import lax
from jax.experimental import pallas as pl
from jax.experimental.pallas import tpu as pltpu
from jax.experimental.pallas import tpu_sc as plsc

HEADS = 6


def _a1_body(x_ref, w_ref, o_ref):
    o_ref[...] = jax.lax.dot_general(
        w_ref[...].astype(jnp.bfloat16), x_ref[...].astype(jnp.bfloat16),
        (((1,), (0,)), ((), ())),
        preferred_element_type=jnp.float32).astype(jnp.bfloat16)


def _dwconv_flat(x, wdw, w_img):
    x = x.astype(jnp.bfloat16).astype(jnp.float32)
    c, hw = x.shape
    col = jax.lax.broadcasted_iota(jnp.int32, (1, hw), 1) % w_img
    mask_l = (col != 0).astype(x.dtype)
    mask_r = (col != (w_img - 1)).astype(x.dtype)
    zero_cache = {}

    def shifted(s):
        if s == 0:
            return x
        if s > 0:
            if s not in zero_cache:
                zero_cache[s] = jnp.zeros((c, s), x.dtype)
            return jnp.concatenate([x[:, s:], zero_cache[s]], axis=1)
        if -s not in zero_cache:
            zero_cache[-s] = jnp.zeros((c, -s), x.dtype)
        return jnp.concatenate([zero_cache[-s], x[:, :s]], axis=1)

    out = None
    for di in (-1, 0, 1):
        for dj in (-1, 0, 1):
            t = wdw[:, (di + 1) * 3 + (dj + 1)][:, None] * shifted(di * w_img + dj)
            if dj == 1:
                t = t * mask_r
            elif dj == -1:
                t = t * mask_l
            out = t if out is None else out + t
    return out


def _dw_tile(w_img, halo, cur_ref, prev_ref, nxt_ref, wdw_ref):
    tt = pl.program_id(2)
    ntt = pl.num_programs(2)
    nt = cur_ref.shape[1]
    mp = jnp.where(tt > 0, 1.0, 0.0).astype(jnp.float32)
    mn = jnp.where(tt < ntt - 1, 1.0, 0.0).astype(jnp.float32)
    x_ext = jnp.concatenate(
        [prev_ref[...].astype(jnp.float32) * mp,
         cur_ref[...].astype(jnp.float32),
         nxt_ref[...].astype(jnp.float32) * mn], axis=1)
    dw = _dwconv_flat(x_ext, wdw_ref[...], w_img)
    return dw[:, halo:halo + nt]


def _a2a_body(w_img, halo, qk_ref, prev_ref, nxt_ref, wdw_ref,
              dw_ref, ssq_ref):
    qk = _dw_tile(w_img, halo, qk_ref, prev_ref, nxt_ref, wdw_ref)
    cc = qk.shape[0] // 2
    q = qk[:cc]
    k = qk[cc:]
    dw_ref[...] = qk
    s = jnp.concatenate(
        [jnp.sum(q * q, axis=1)[None, :], jnp.sum(k * k, axis=1)[None, :]],
        axis=0)
    tt = pl.program_id(2)

    @pl.when(tt == 0)
    def _():
        ssq_ref[...] = s

    @pl.when(tt > 0)
    def _():
        ssq_ref[...] += s


def _a2b_body(qk_ref, rn_ref, gram_ref):
    cc = qk_ref.shape[0] // 2
    qn = (qk_ref[:cc] * rn_ref[0]).astype(jnp.bfloat16)
    kn = (qk_ref[cc:] * rn_ref[1]).astype(jnp.bfloat16)
    g = jax.lax.dot_general(
        qn, kn, (((1,), (1,)), ((), ())), preferred_element_type=jnp.float32)
    tt = pl.program_id(2)

    @pl.when(tt == 0)
    def _():
        gram_ref[...] = g

    @pl.when(tt > 0)
    def _():
        gram_ref[...] += g


def _a2bv_body(w_img, halo, qk_ref, rn_ref, v_ref, prev_ref, nxt_ref,
               wdw_ref, gram_ref, o_ref):
    o_ref[...] = _dw_tile(w_img, halo, v_ref, prev_ref, nxt_ref,
                          wdw_ref).astype(jnp.bfloat16)
    cc = qk_ref.shape[0] // 2
    qn = (qk_ref[:cc] * rn_ref[0]).astype(jnp.bfloat16)
    kn = (qk_ref[cc:] * rn_ref[1]).astype(jnp.bfloat16)
    g = jax.lax.dot_general(
        qn, kn, (((1,), (1,)), ((), ())), preferred_element_type=jnp.float32)
    tt = pl.program_id(2)

    @pl.when(tt == 0)
    def _():
        gram_ref[...] = g

    @pl.when(tt > 0)
    def _():
        gram_ref[...] += g


def _spa_body(ks, gram_ref, ssq_ref, temp_ref, coef_ref, spa_ref):
    bh = gram_ref.shape[0]
    cc = gram_ref.shape[1]
    n = cc * cc
    nq = jnp.maximum(jnp.sqrt(ssq_ref[:, 0, :]), 1e-12)
    nk = jnp.maximum(jnp.sqrt(ssq_ref[:, 1, :]), 1e-12)
    attn = gram_ref[...] / (nq[:, :, None] * nk[:, None, :]) * temp_ref[...]

    bits = jax.lax.bitcast_convert_type(attn, jnp.int32)
    skey = jnp.where(bits < 0, bits ^ jnp.int32(0x7FFFFFFF), bits)

    klist = (n * 1 // 2, n * 2 // 3, n * 3 // 4, n * 4 // 5)
    m = jnp.max(attn, axis=(1, 2), keepdims=True)
    e = jnp.exp(attn - m)
    coeff = jnp.zeros_like(attn)
    for ki, kk in enumerate(klist):
        def bit_step(i, t_u, kk=kk):
            b = 31 - i
            t_try = t_u | (jnp.int32(1) << b)
            t_cmp = t_try ^ jnp.int32(-0x80000000)
            cnt = jnp.sum((skey >= t_cmp[:, :, None]).astype(jnp.int32),
                          axis=(1, 2), keepdims=True)[:, :, 0]
            return jnp.where(cnt >= kk, t_try, t_u)

        t_u = jax.lax.fori_loop(0, 32, bit_step,
                                jnp.zeros((bh, 1), jnp.int32))
        th_skey = t_u ^ jnp.int32(-0x80000000)
        th_bits = jnp.where(th_skey < 0, th_skey ^ jnp.int32(0x7FFFFFFF),
                            th_skey)
        th = jax.lax.bitcast_convert_type(th_bits, jnp.float32)
        th3 = th[:, :, None]
        gt = (attn > th3).astype(jnp.float32)
        eq = (attn == th3).astype(jnp.float32)
        c_g = jnp.sum(gt, axis=(1, 2), keepdims=True)
        c_e = jnp.sum(eq, axis=(1, 2), keepdims=True)
        e_th = jnp.exp(th3 - m)
        z = jnp.sum(e * gt, axis=(1, 2), keepdims=True) + (kk - c_g) * e_th
        a = coef_ref[0, ki]
        coeff = coeff + gt * (a / z) + eq * (a * (kk - c_g) / (c_e * z))
    spa_ref[...] = e * coeff


def _xl_reduce(v, op):
    for sh in (8, 4, 2, 1):
        idx = lax.iota(jnp.int32, 16) ^ sh
        v = op(v, v.at[idx].get(mode="promise_in_bounds"))
    return v


def _make_spa_sc(bh, n, klist):
    nv = n // 16
    mesh = plsc.VectorSubcoreMesh(core_axis_name="c", subcore_axis_name="s")

    @functools.partial(
        pl.kernel, mesh=mesh,
        out_type=jax.ShapeDtypeStruct((bh, n), jnp.float32),
        scratch_types=[
            pltpu.VMEM((n,), jnp.float32),
            pltpu.VMEM((n,), jnp.int32),
            pltpu.VMEM((n,), jnp.float32),
            pltpu.VMEM((64,), jnp.float32),
        ],
    )
    def spa_sc(attn_hbm, skey_hbm, coef_hbm, out_hbm,
               attn_v, skey_v, out_v, coef_v):
        i32 = jnp.int32
        wid = lax.axis_index("s") * 2 + lax.axis_index("c")

        @pl.when(wid < bh)
        def _():
            pltpu.sync_copy(attn_hbm.at[wid], attn_v)
            pltpu.sync_copy(skey_hbm.at[wid], skey_v)
            pltpu.sync_copy(coef_hbm, coef_v)
            coefs = [coef_v[pl.ds(ki * 16, 16)] for ki in range(4)]

            def pass1(i, mv):
                return jnp.maximum(mv, attn_v[pl.ds(i * 16, 16)])

            mv = lax.fori_loop(0, nv, pass1,
                               jnp.full((16,), -3.4e38, jnp.float32))
            m_s = _xl_reduce(mv, jnp.maximum)

            ths = []
            for kk in klist:
                def bit_step(bi, t_u, kk=kk):
                    b = 31 - bi
                    t_try = t_u | (i32(1) << b)
                    t_cmp = t_try ^ i32(-0x80000000)

                    def count(i, cnt):
                        sv = skey_v[pl.ds(i * 16, 16)]
                        return cnt + jnp.where(sv >= t_cmp, 1, 0)

                    cnt = _xl_reduce(
                        lax.fori_loop(0, nv, count, jnp.zeros((16,), i32)),
                        jnp.add)
                    return jnp.where(cnt >= kk, t_try, t_u)

                t_u = lax.fori_loop(0, 32, bit_step, jnp.zeros((16,), i32))
                ths.append(t_u ^ i32(-0x80000000))

            def pass2(i, acc):
                a = attn_v[pl.ds(i * 16, 16)]
                sv = skey_v[pl.ds(i * 16, 16)]
                e = jnp.exp(a - m_s)
                out = []
                for ki in range(4):
                    z, zeq, cg, ce = acc[ki]
                    gt = sv > ths[ki]
                    eq = sv == ths[ki]
                    out.append((z + jnp.where(gt, e, 0.0),
                                zeq + jnp.where(eq, e, 0.0),
                                cg + jnp.where(gt, 1, 0),
                                ce + jnp.where(eq, 1, 0)))
                return tuple(out)

            zero = (jnp.zeros((16,), jnp.float32),
                    jnp.zeros((16,), jnp.float32),
                    jnp.zeros((16,), i32), jnp.zeros((16,), i32))
            acc = lax.fori_loop(0, nv, pass2, (zero, zero, zero, zero))

            wks = []
            for ki, kk in enumerate(klist):
                z, zeq, cg, ce = acc[ki]
                z_tot = _xl_reduce(z, jnp.add)
                zeq_tot = _xl_reduce(zeq, jnp.add)
                cgf = _xl_reduce(cg, jnp.add).astype(jnp.float32)
                cef = _xl_reduce(ce, jnp.add).astype(jnp.float32)
                e_th = zeq_tot / cef
                zz = z_tot + (kk - cgf) * e_th
                wks.append((coefs[ki] / zz,
                            coefs[ki] * (kk - cgf) / (cef * zz)))

            def pass3(i, carry):
                a = attn_v[pl.ds(i * 16, 16)]
                sv = skey_v[pl.ds(i * 16, 16)]
                e = jnp.exp(a - m_s)
                coeff = jnp.zeros((16,), jnp.float32)
                for ki in range(4):
                    coeff = coeff + jnp.where(sv > ths[ki], wks[ki][0], 0.0)
                    coeff = coeff + jnp.where(sv == ths[ki], wks[ki][1], 0.0)
                out_v[pl.ds(i * 16, 16)] = e * coeff
                return carry

            lax.fori_loop(0, nv, pass3, 0)
            pltpu.sync_copy(out_v, out_hbm.at[wid])

    return spa_sc


def _c_body(heads, gram_like_spa_ref, wpo_ref, v_ref, o_ref):
    spa = gram_like_spa_ref[...]
    cc = spa.shape[1]
    wpo = wpo_ref[...]
    cols = []
    for h in range(heads):
        cols.append(jax.lax.dot_general(
            wpo[:, h * cc:(h + 1) * cc].astype(jnp.bfloat16),
            spa[h].astype(jnp.bfloat16), (((1,), (0,)), ((), ())),
            preferred_element_type=jnp.float32))
    mmat = jnp.concatenate(cols, axis=1)
    o_ref[...] = jax.lax.dot_general(
        mmat.astype(jnp.bfloat16), v_ref[...],
        (((1,), (0,)), ((), ())),
        preferred_element_type=jnp.float32)


def _pick_tiles(hw):
    for nt in (14, 8, 7, 4, 2):
        if hw % nt == 0:
            return nt
    return 1


def _plan_dw_tiles(hw, w_img):
    halo = w_img
    while halo % 128 != 0 or halo <= w_img:
        halo += w_img
    best = None
    for ntile in range(1, 64):
        if hw % ntile:
            continue
        nt = hw // ntile
        if nt % halo:
            continue
        if nt * 64 * 4 <= 4 * 1024 * 1024 or best is None:
            best = (ntile, nt, halo)
            if nt * 64 * 4 <= 4 * 1024 * 1024:
                return best
    return best


def kernel(x_in, W_qkv, W_dw, W_po, temperature, attn1, attn2, attn3, attn4):
    b, dim, h_img, w_img = x_in.shape
    heads = HEADS
    cc = dim // heads
    hw = h_img * w_img
    f32 = jnp.float32

    base = jnp.arange(cc)
    perm = []
    for h in range(heads):
        perm.append(h * cc + base)
        perm.append(dim + h * cc + base)
    for h in range(heads):
        perm.append(2 * dim + h * cc + base)
    perm = jnp.concatenate(perm)

    w1 = W_qkv[:, :, 0, 0][perm]
    wdw = W_dw[:, 0].reshape(3 * dim, 9)[perm]
    wpo = W_po[:, :, 0, 0]

    x = x_in.reshape(b, dim, hw)
    ntile = _pick_tiles(hw)
    nt = hw // ntile

    qkv_pre = pl.pallas_call(
        _a1_body,
        grid=(b, ntile),
        in_specs=[
            pl.BlockSpec((None, dim, nt), lambda bb, tt: (bb, 0, tt)),
            pl.BlockSpec((3 * dim, dim), lambda bb, tt: (0, 0)),
        ],
        out_specs=pl.BlockSpec((None, 3 * dim, nt), lambda bb, tt: (bb, 0, tt)),
        out_shape=jax.ShapeDtypeStruct((b, 3 * dim, hw), jnp.bfloat16),
    )(x, w1)

    ntile2, nt2, halo = _plan_dw_tiles(hw, w_img)
    rr = nt2 // halo
    nhalo = hw // halo

    def _prev_idx(bb, hh, tt):
        return (bb, hh, jnp.maximum(tt * rr - 1, 0))

    def _nxt_idx(bb, hh, tt):
        return (bb, hh, jnp.minimum((tt + 1) * rr, nhalo - 1))

    qkdw, ssq = pl.pallas_call(
        functools.partial(_a2a_body, w_img, halo),
        grid=(b, heads, ntile2),
        in_specs=[
            pl.BlockSpec((None, 2 * cc, nt2), lambda bb, hh, tt: (bb, hh, tt)),
            pl.BlockSpec((None, 2 * cc, halo), _prev_idx),
            pl.BlockSpec((None, 2 * cc, halo), _nxt_idx),
            pl.BlockSpec((2 * cc, 9), lambda bb, hh, tt: (hh, 0)),
        ],
        out_specs=[
            pl.BlockSpec((None, 2 * cc, nt2), lambda bb, hh, tt: (bb, hh, tt)),
            pl.BlockSpec((None, 2, cc),
                         lambda bb, hh, tt: (bb * heads + hh, 0, 0)),
        ],
        out_shape=[
            jax.ShapeDtypeStruct((b, 2 * cc * heads, hw), f32),
            jax.ShapeDtypeStruct((b * heads, 2, cc), f32),
        ],
    )(qkv_pre, qkv_pre, qkv_pre, wdw)

    rnorm = (1.0 / jnp.maximum(jnp.sqrt(ssq), 1e-12))[..., None]

    def _prev_idx_v(bb, hh, tt):
        return (bb, 2 * heads + hh, jnp.maximum(tt * rr - 1, 0))

    def _nxt_idx_v(bb, hh, tt):
        return (bb, 2 * heads + hh, jnp.minimum((tt + 1) * rr, nhalo - 1))

    gram, v = pl.pallas_call(
        functools.partial(_a2bv_body, w_img, halo),
        grid=(b, heads, ntile2),
        in_specs=[
            pl.BlockSpec((None, 2 * cc, nt2), lambda bb, hh, tt: (bb, hh, tt)),
            pl.BlockSpec((None, 2, cc, 1),
                         lambda bb, hh, tt: (bb * heads + hh, 0, 0, 0)),
            pl.BlockSpec((None, cc, nt2),
                         lambda bb, hh, tt: (bb, 2 * heads + hh, tt)),
            pl.BlockSpec((None, cc, halo), _prev_idx_v),
            pl.BlockSpec((None, cc, halo), _nxt_idx_v),
            pl.BlockSpec((cc, 9), lambda bb, hh, tt: (2 * heads + hh, 0)),
        ],
        out_specs=[
            pl.BlockSpec((None, cc, cc),
                         lambda bb, hh, tt: (bb * heads + hh, 0, 0)),
            pl.BlockSpec((None, None, cc, nt2),
                         lambda bb, hh, tt: (bb, hh, 0, tt)),
        ],
        out_shape=[
            jax.ShapeDtypeStruct((b * heads, cc, cc), f32),
            jax.ShapeDtypeStruct((b, heads, cc, hw), jnp.bfloat16),
        ],
    )(qkdw, rnorm, qkv_pre, qkv_pre, qkv_pre, wdw)

    temp_b = jnp.broadcast_to(temperature[None, :, :, :],
                              (b, heads, 1, 1)).reshape(b * heads, 1, 1)
    coef = jnp.concatenate([attn1, attn2, attn3, attn4])
    attn_rows = (gram * temp_b).reshape(b * heads, cc * cc)
    klist = (cc * cc * 1 // 2, cc * cc * 2 // 3, cc * cc * 3 // 4,
             cc * cc * 4 // 5)
    coef_bc = jnp.broadcast_to(coef[:, None], (4, 16)).reshape(64)
    bits = jax.lax.bitcast_convert_type(attn_rows, jnp.int32)
    skey_rows = jnp.where(bits < 0, bits ^ jnp.int32(0x7FFFFFFF), bits)
    spa = _make_spa_sc(b * heads, cc * cc, klist)(
        attn_rows, skey_rows, coef_bc)

    v2 = v.reshape(b, dim, hw)
    spa_b = spa.reshape(b, heads, cc, cc)
    out = pl.pallas_call(
        functools.partial(_c_body, heads),
        grid=(b, ntile),
        in_specs=[
            pl.BlockSpec((None, heads, cc, cc), lambda bb, tt: (bb, 0, 0, 0)),
            pl.BlockSpec((dim, dim), lambda bb, tt: (0, 0)),
            pl.BlockSpec((None, dim, nt), lambda bb, tt: (bb, 0, tt)),
        ],
        out_specs=pl.BlockSpec((None, dim, nt), lambda bb, tt: (bb, 0, tt)),
        out_shape=jax.ShapeDtypeStruct((b, dim, hw), f32),
    )(spa_b, wpo, v2)

    return out.reshape(b, dim, h_img, w_img)
